# baseline (speedup 1.0000x reference)
import functools

import jax
import jax.numpy as jnp
from jax import lax
from jax.experimental import pallas as pl
from jax.experimental.pallas import tpu as pltpu
from jax.experimental.pallas import tpu_sc as plsc

N = 10000
E = 320000
C = 40
C_PAD = 48

NC = 2
NS = 16
NW = NC * NS
B = 128
EPW = 10112
E_PAD = EPW * NW
N_PAD = 10240
CH_A = 113
CH_B = 2 * (EPW // B) - CH_A
RPT = N_PAD // NS
DEG_D = 16


def _mesh():
    return plsc.VectorSubcoreMesh(core_axis_name="c", subcore_axis_name="s")


_SC_PARAMS = pltpu.CompilerParams(use_tc_tiling_on_sc=False)


@functools.partial(
    pl.kernel,
    out_type=jax.ShapeDtypeStruct((NC, N_PAD, DEG_D), jnp.float32),
    mesh=_mesh(),
    compiler_params=_SC_PARAMS,
    scratch_types=[
        pltpu.VMEM((EPW // B, B), jnp.int32),
        pltpu.VMEM((B, DEG_D), jnp.float32),
        pltpu.VMEM((B, DEG_D), jnp.float32),
        pltpu.VMEM_SHARED((N_PAD, DEG_D), jnp.float32),
    ],
)
def _deg_kernel(dstp_hbm, out_hbm, dst_all, ones_v, zero_v, acc_sh):
    cid = lax.axis_index("c")
    sid = lax.axis_index("s")
    wid = cid * NS + sid

    one16 = jnp.ones((DEG_D,), jnp.float32)
    z16 = jnp.zeros((DEG_D,), jnp.float32)

    def fill(i, _):
        ones_v[i, :] = one16
        zero_v[i, :] = z16
        return 0

    lax.fori_loop(0, B, fill, 0)
    pltpu.sync_copy(dstp_hbm.at[wid], dst_all)
    for k in range(RPT // B):
        pltpu.sync_copy(zero_v, acc_sh.at[pl.ds(sid * RPT + k * B, B)])
    plsc.subcore_barrier()

    def body(i, _):
        pltpu.sync_copy(ones_v, acc_sh.at[dst_all.at[i]], add=True)
        return 0

    lax.fori_loop(0, EPW // B, body, 0)
    plsc.subcore_barrier()
    pltpu.sync_copy(
        acc_sh.at[pl.ds(sid * RPT, RPT)], out_hbm.at[cid, pl.ds(sid * RPT, RPT)]
    )


def _make_agg(D):
    @functools.partial(
        pl.kernel,
        out_type=jax.ShapeDtypeStruct((NC, N_PAD, D), jnp.float32),
        mesh=_mesh(),
        compiler_params=_SC_PARAMS,
        scratch_types=[
            pltpu.VMEM((max(CH_A, CH_B) * B,), jnp.int32),
            pltpu.VMEM((B,), jnp.int32),
            pltpu.VMEM((B,), jnp.int32),
            pltpu.VMEM((B, D), jnp.float32),
            pltpu.VMEM((B, D), jnp.float32),
            pltpu.SemaphoreType.DMA,
            pltpu.SemaphoreType.DMA,
            pltpu.VMEM_SHARED((N_PAD, D), jnp.float32),
        ],
    )
    def agg(g_hbm, srcp_hbm, dstp_hbm, out_hbm, src_all, dst0, dst1,
            rows0, rows1, sem0, sem1, acc_sh):
        cid = lax.axis_index("c")
        sid = lax.axis_index("s")
        wid = cid * NS + sid
        base0 = wid * EPW

        z16 = jnp.zeros((16,), jnp.float32)
        dl = D // 16

        def zfill(i, _):
            rows0[i // dl, pl.ds((i % dl) * 16, 16)] = z16
            return 0

        lax.fori_loop(0, B * dl, zfill, 0)
        for k in range(RPT // B):
            pltpu.sync_copy(rows0, acc_sh.at[pl.ds(sid * RPT + k * B, B)])
        plsc.subcore_barrier()

        def run_pipe(nch, base0):
            epw = nch * B
            pltpu.sync_copy(
                srcp_hbm.at[pl.ds(base0, epw)], src_all.at[pl.ds(0, epw)]
                )
            pltpu.async_copy(g_hbm.at[src_all.at[pl.ds(0, B)]], rows0, sem0)
            pltpu.async_copy(g_hbm.at[src_all.at[pl.ds(B, B)]], rows1, sem1)

            def body(j, _):
                i0 = base0 + 2 * j * B
                o0 = 2 * j * B
                pltpu.sync_copy(dstp_hbm.at[pl.ds(i0, B)], dst0)
                pltpu.make_async_copy(
                    g_hbm.at[src_all.at[pl.ds(o0, B)]], rows0, sem0
                ).wait()
                pltpu.sync_copy(rows0, acc_sh.at[dst0], add=True)
                pltpu.async_copy(
                    g_hbm.at[src_all.at[pl.ds(o0 + 2 * B, B)]], rows0, sem0
                )
                pltpu.sync_copy(dstp_hbm.at[pl.ds(i0 + B, B)], dst1)
                pltpu.make_async_copy(
                    g_hbm.at[src_all.at[pl.ds(o0 + B, B)]], rows1, sem1
                ).wait()
                pltpu.sync_copy(rows1, acc_sh.at[dst1], add=True)
                pltpu.async_copy(
                    g_hbm.at[src_all.at[pl.ds(o0 + 3 * B, B)]], rows1, sem1
                )
                return 0

            lax.fori_loop(0, (nch - 3) // 2, body, 0)
            pltpu.sync_copy(dstp_hbm.at[pl.ds(base0 + epw - 3 * B, B)], dst0)
            pltpu.make_async_copy(
                g_hbm.at[src_all.at[pl.ds(epw - 3 * B, B)]], rows0, sem0
            ).wait()
            pltpu.sync_copy(rows0, acc_sh.at[dst0], add=True)
            pltpu.sync_copy(dstp_hbm.at[pl.ds(base0 + epw - 2 * B, B)], dst1)
            pltpu.make_async_copy(
                g_hbm.at[src_all.at[pl.ds(epw - 2 * B, B)]], rows1, sem1
            ).wait()
            pltpu.sync_copy(rows1, acc_sh.at[dst1], add=True)
            pltpu.async_copy(g_hbm.at[src_all.at[pl.ds(epw - B, B)]], rows0, sem0)
            pltpu.sync_copy(dstp_hbm.at[pl.ds(base0 + epw - B, B)], dst0)
            pltpu.make_async_copy(
                g_hbm.at[src_all.at[pl.ds(epw - B, B)]], rows0, sem0
            ).wait()
            pltpu.sync_copy(rows0, acc_sh.at[dst0], add=True)

        @pl.when(cid == 0)
        def _():
            run_pipe(CH_A, sid * CH_A * B)

        @pl.when(cid == 1)
        def _():
            run_pipe(CH_B, NS * CH_A * B + sid * CH_B * B)

        plsc.subcore_barrier()
        pltpu.sync_copy(
            acc_sh.at[pl.ds(sid * RPT, RPT)], out_hbm.at[cid, pl.ds(sid * RPT, RPT)]
        )

    return agg


_agg128 = _make_agg(128)
_agg64 = _make_agg(64)
_agg48 = _make_agg(C_PAD)


def _tc_first_body(deg_ref, x_ref, w_ref, d_ref, g_ref):
    deg = deg_ref[0] + deg_ref[1] + 1.0
    d = lax.rsqrt(deg)
    d_ref[...] = d
    g_ref[...] = d * jnp.dot(
        x_ref[...], w_ref[...], preferred_element_type=jnp.float32
    )


def _tc_mid_body(acc_ref, g_ref, d_ref, b_ref, w_ref, go_ref):
    a = acc_ref[0, 0:N, :] + acc_ref[1, 0:N, :] + g_ref[...]
    d = d_ref[...]
    h = jnp.maximum(d * a + b_ref[...], 0.0)
    go_ref[...] = d * jnp.dot(h, w_ref[...], preferred_element_type=jnp.float32)


def _tc_out_body(acc_ref, g_ref, d_ref, b_ref, o_ref):
    a = acc_ref[0, 0:N, :] + acc_ref[1, 0:N, :] + g_ref[...]
    o_ref[...] = d_ref[...] * a + b_ref[...]


def _tc_first(degs, x, W1):
    return pl.pallas_call(
        _tc_first_body,
        out_shape=[
            jax.ShapeDtypeStruct((N, 1), jnp.float32),
            jax.ShapeDtypeStruct((N, W1.shape[1]), jnp.float32),
        ],
    )(degs, x, W1)


def _tc_mid(acc, g, d, b, W):
    return pl.pallas_call(
        _tc_mid_body,
        out_shape=jax.ShapeDtypeStruct((N, W.shape[1]), jnp.float32),
    )(acc, g, d, b, W)


def _tc_out(acc, g, d, b):
    return pl.pallas_call(
        _tc_out_body,
        out_shape=jax.ShapeDtypeStruct((N, g.shape[1]), jnp.float32),
    )(acc, g, d, b)


def kernel(x, edge_index, W1, b1, W2, b2, W3, b3):
    src = edge_index[0].astype(jnp.int32)
    dst = edge_index[1].astype(jnp.int32)
    pad = E_PAD - E
    srcp = jnp.concatenate([src, jnp.zeros((pad,), jnp.int32)])
    pad_dst = N + (jnp.arange(pad, dtype=jnp.int32) % (N_PAD - N))
    dstp = jnp.concatenate([dst, pad_dst])

    deg_p = _deg_kernel(dstp.reshape(NW, EPW // B, B))
    degs = deg_p[:, :N, 0:1]

    d, g1 = _tc_first(degs, x, W1)
    acc1 = _agg128(g1, srcp, dstp)
    g2 = _tc_mid(acc1, g1, d, b1.reshape(1, -1), W2)
    acc2 = _agg64(g2, srcp, dstp)
    W3p = jnp.pad(W3, ((0, 0), (0, C_PAD - C)))
    b3p = jnp.pad(b3, (0, C_PAD - C)).reshape(1, -1)
    g3 = _tc_mid(acc2, g2, d, b2.reshape(1, -1), W3p)
    acc3 = _agg48(g3, srcp, dstp)
    out = _tc_out(acc3, g3, d, b3p)
    return out[:, :C]


# speedup vs baseline: 1.0012x; 1.0012x over previous
import functools

import jax
import jax.numpy as jnp
from jax import lax
from jax.experimental import pallas as pl
from jax.experimental.pallas import tpu as pltpu
from jax.experimental.pallas import tpu_sc as plsc

N = 10000
E = 320000
C = 40
C_PAD = 48

NC = 2
NS = 16
NW = NC * NS
B = 128
EPW = 10112
E_PAD = EPW * NW
N_PAD = 10240
CH_A = 109
CH_B = 2 * (EPW // B) - CH_A
RPT = N_PAD // NS
DEG_D = 16


def _mesh():
    return plsc.VectorSubcoreMesh(core_axis_name="c", subcore_axis_name="s")


_SC_PARAMS = pltpu.CompilerParams(use_tc_tiling_on_sc=False)


@functools.partial(
    pl.kernel,
    out_type=jax.ShapeDtypeStruct((NC, N_PAD, DEG_D), jnp.float32),
    mesh=_mesh(),
    compiler_params=_SC_PARAMS,
    scratch_types=[
        pltpu.VMEM((EPW // B, B), jnp.int32),
        pltpu.VMEM((B, DEG_D), jnp.float32),
        pltpu.VMEM((B, DEG_D), jnp.float32),
        pltpu.VMEM_SHARED((N_PAD, DEG_D), jnp.float32),
    ],
)
def _deg_kernel(dstp_hbm, out_hbm, dst_all, ones_v, zero_v, acc_sh):
    cid = lax.axis_index("c")
    sid = lax.axis_index("s")
    wid = cid * NS + sid

    one16 = jnp.ones((DEG_D,), jnp.float32)
    z16 = jnp.zeros((DEG_D,), jnp.float32)

    def fill(i, _):
        ones_v[i, :] = one16
        zero_v[i, :] = z16
        return 0

    lax.fori_loop(0, B, fill, 0)
    pltpu.sync_copy(dstp_hbm.at[wid], dst_all)
    for k in range(RPT // B):
        pltpu.sync_copy(zero_v, acc_sh.at[pl.ds(sid * RPT + k * B, B)])
    plsc.subcore_barrier()

    def body(i, _):
        pltpu.sync_copy(ones_v, acc_sh.at[dst_all.at[i]], add=True)
        return 0

    lax.fori_loop(0, EPW // B, body, 0)
    plsc.subcore_barrier()
    pltpu.sync_copy(
        acc_sh.at[pl.ds(sid * RPT, RPT)], out_hbm.at[cid, pl.ds(sid * RPT, RPT)]
    )


def _make_agg(D):
    @functools.partial(
        pl.kernel,
        out_type=jax.ShapeDtypeStruct((NC, N_PAD, D), jnp.float32),
        mesh=_mesh(),
        compiler_params=_SC_PARAMS,
        scratch_types=[
            pltpu.VMEM((max(CH_A, CH_B) * B,), jnp.int32),
            pltpu.VMEM((B,), jnp.int32),
            pltpu.VMEM((B,), jnp.int32),
            pltpu.VMEM((B, D), jnp.float32),
            pltpu.VMEM((B, D), jnp.float32),
            pltpu.SemaphoreType.DMA,
            pltpu.SemaphoreType.DMA,
            pltpu.VMEM_SHARED((N_PAD, D), jnp.float32),
        ],
    )
    def agg(g_hbm, srcp_hbm, dstp_hbm, out_hbm, src_all, dst0, dst1,
            rows0, rows1, sem0, sem1, acc_sh):
        cid = lax.axis_index("c")
        sid = lax.axis_index("s")
        wid = cid * NS + sid
        base0 = wid * EPW

        z16 = jnp.zeros((16,), jnp.float32)
        dl = D // 16

        def zfill(i, _):
            rows0[i // dl, pl.ds((i % dl) * 16, 16)] = z16
            return 0

        lax.fori_loop(0, B * dl, zfill, 0)
        for k in range(RPT // B):
            pltpu.sync_copy(rows0, acc_sh.at[pl.ds(sid * RPT + k * B, B)])
        plsc.subcore_barrier()

        def run_pipe(nch, base0):
            epw = nch * B
            pltpu.sync_copy(
                srcp_hbm.at[pl.ds(base0, epw)], src_all.at[pl.ds(0, epw)]
                )
            pltpu.async_copy(g_hbm.at[src_all.at[pl.ds(0, B)]], rows0, sem0)
            pltpu.async_copy(g_hbm.at[src_all.at[pl.ds(B, B)]], rows1, sem1)

            def body(j, _):
                i0 = base0 + 2 * j * B
                o0 = 2 * j * B
                pltpu.sync_copy(dstp_hbm.at[pl.ds(i0, B)], dst0)
                pltpu.make_async_copy(
                    g_hbm.at[src_all.at[pl.ds(o0, B)]], rows0, sem0
                ).wait()
                pltpu.sync_copy(rows0, acc_sh.at[dst0], add=True)
                pltpu.async_copy(
                    g_hbm.at[src_all.at[pl.ds(o0 + 2 * B, B)]], rows0, sem0
                )
                pltpu.sync_copy(dstp_hbm.at[pl.ds(i0 + B, B)], dst1)
                pltpu.make_async_copy(
                    g_hbm.at[src_all.at[pl.ds(o0 + B, B)]], rows1, sem1
                ).wait()
                pltpu.sync_copy(rows1, acc_sh.at[dst1], add=True)
                pltpu.async_copy(
                    g_hbm.at[src_all.at[pl.ds(o0 + 3 * B, B)]], rows1, sem1
                )
                return 0

            lax.fori_loop(0, (nch - 3) // 2, body, 0)
            pltpu.sync_copy(dstp_hbm.at[pl.ds(base0 + epw - 3 * B, B)], dst0)
            pltpu.make_async_copy(
                g_hbm.at[src_all.at[pl.ds(epw - 3 * B, B)]], rows0, sem0
            ).wait()
            pltpu.sync_copy(rows0, acc_sh.at[dst0], add=True)
            pltpu.sync_copy(dstp_hbm.at[pl.ds(base0 + epw - 2 * B, B)], dst1)
            pltpu.make_async_copy(
                g_hbm.at[src_all.at[pl.ds(epw - 2 * B, B)]], rows1, sem1
            ).wait()
            pltpu.sync_copy(rows1, acc_sh.at[dst1], add=True)
            pltpu.async_copy(g_hbm.at[src_all.at[pl.ds(epw - B, B)]], rows0, sem0)
            pltpu.sync_copy(dstp_hbm.at[pl.ds(base0 + epw - B, B)], dst0)
            pltpu.make_async_copy(
                g_hbm.at[src_all.at[pl.ds(epw - B, B)]], rows0, sem0
            ).wait()
            pltpu.sync_copy(rows0, acc_sh.at[dst0], add=True)

        @pl.when(cid == 0)
        def _():
            run_pipe(CH_A, sid * CH_A * B)

        @pl.when(cid == 1)
        def _():
            run_pipe(CH_B, NS * CH_A * B + sid * CH_B * B)

        plsc.subcore_barrier()
        pltpu.sync_copy(
            acc_sh.at[pl.ds(sid * RPT, RPT)], out_hbm.at[cid, pl.ds(sid * RPT, RPT)]
        )

    return agg


_agg128 = _make_agg(128)
_agg64 = _make_agg(64)
_agg48 = _make_agg(C_PAD)


def _tc_first_body(deg_ref, x_ref, w_ref, d_ref, g_ref):
    deg = deg_ref[0] + deg_ref[1] + 1.0
    d = lax.rsqrt(deg)
    d_ref[...] = d
    g_ref[...] = d * jnp.dot(
        x_ref[...], w_ref[...], preferred_element_type=jnp.float32
    )


def _tc_mid_body(acc_ref, g_ref, d_ref, b_ref, w_ref, go_ref):
    a = acc_ref[0, 0:N, :] + acc_ref[1, 0:N, :] + g_ref[...]
    d = d_ref[...]
    h = jnp.maximum(d * a + b_ref[...], 0.0)
    go_ref[...] = d * jnp.dot(h, w_ref[...], preferred_element_type=jnp.float32)


def _tc_out_body(acc_ref, g_ref, d_ref, b_ref, o_ref):
    a = acc_ref[0, 0:N, :] + acc_ref[1, 0:N, :] + g_ref[...]
    o_ref[...] = d_ref[...] * a + b_ref[...]


def _tc_first(degs, x, W1):
    return pl.pallas_call(
        _tc_first_body,
        out_shape=[
            jax.ShapeDtypeStruct((N, 1), jnp.float32),
            jax.ShapeDtypeStruct((N, W1.shape[1]), jnp.float32),
        ],
    )(degs, x, W1)


def _tc_mid(acc, g, d, b, W):
    return pl.pallas_call(
        _tc_mid_body,
        out_shape=jax.ShapeDtypeStruct((N, W.shape[1]), jnp.float32),
    )(acc, g, d, b, W)


def _tc_out(acc, g, d, b):
    return pl.pallas_call(
        _tc_out_body,
        out_shape=jax.ShapeDtypeStruct((N, g.shape[1]), jnp.float32),
    )(acc, g, d, b)


def kernel(x, edge_index, W1, b1, W2, b2, W3, b3):
    src = edge_index[0].astype(jnp.int32)
    dst = edge_index[1].astype(jnp.int32)
    pad = E_PAD - E
    srcp = jnp.concatenate([src, jnp.zeros((pad,), jnp.int32)])
    pad_dst = N + (jnp.arange(pad, dtype=jnp.int32) % (N_PAD - N))
    dstp = jnp.concatenate([dst, pad_dst])

    deg_p = _deg_kernel(dstp.reshape(NW, EPW // B, B))
    degs = deg_p[:, :N, 0:1]

    d, g1 = _tc_first(degs, x, W1)
    acc1 = _agg128(g1, srcp, dstp)
    g2 = _tc_mid(acc1, g1, d, b1.reshape(1, -1), W2)
    acc2 = _agg64(g2, srcp, dstp)
    W3p = jnp.pad(W3, ((0, 0), (0, C_PAD - C)))
    b3p = jnp.pad(b3, (0, C_PAD - C)).reshape(1, -1)
    g3 = _tc_mid(acc2, g2, d, b2.reshape(1, -1), W3p)
    acc3 = _agg48(g3, srcp, dstp)
    out = _tc_out(acc3, g3, d, b3p)
    return out[:, :C]
